# Initial kernel scaffold; baseline (speedup 1.0000x reference)
import jax
import jax.numpy as jnp
from jax.experimental import pallas as pl


def kernel(x, emb_table, seq_len, front, back, keyframe_gap):
    raise NotImplementedError("write your pallas kernel here")


# speedup vs baseline: 1.3832x; 1.3832x over previous
import jax
import jax.numpy as jnp
from jax.experimental import pallas as pl
from jax.experimental.pallas import tpu as pltpu

_SEQ_BLK = 512


def _body(bounds_ref, x_ref, emb_ref, o_ref):
    j = pl.program_id(1)
    pos = jax.lax.broadcasted_iota(jnp.int32, (_SEQ_BLK, 1), 0) + j * _SEQ_BLK
    unknown_start = bounds_ref[0]
    back_start = bounds_ref[1]
    ignored_start = bounds_ref[2]
    t0 = emb_ref[0:1, :]
    t1 = emb_ref[1:2, :]
    t2 = emb_ref[2:3, :]
    emb = jnp.where(
        pos < unknown_start,
        t0,
        jnp.where(pos < back_start, t1, jnp.where(pos < ignored_start, t0, t2)),
    )
    o_ref[...] = x_ref[0] + emb


def kernel(x, emb_table, seq_len, front, back, keyframe_gap):
    batch, n, d = x.shape
    seq_len = jnp.asarray(seq_len, jnp.int32)
    front = jnp.asarray(front, jnp.int32)
    back = jnp.asarray(back, jnp.int32)
    keyframe_gap = jnp.asarray(keyframe_gap, jnp.int32)
    ignored_len = seq_len - front - back - keyframe_gap
    bounds = jnp.stack(
        [front, front + keyframe_gap, seq_len - ignored_len], axis=0
    ).astype(jnp.int32)

    grid = (batch, n // _SEQ_BLK)
    return pl.pallas_call(
        _body,
        grid=grid,
        in_specs=[
            pl.BlockSpec(memory_space=pltpu.SMEM),
            pl.BlockSpec((1, _SEQ_BLK, d), lambda b, j: (b, j, 0)),
            pl.BlockSpec((3, d), lambda b, j: (0, 0)),
        ],
        out_specs=pl.BlockSpec((_SEQ_BLK, d), lambda b, j: (b * (n // _SEQ_BLK) + j, 0)),
        out_shape=jax.ShapeDtypeStruct((batch * n, d), x.dtype),
    )(bounds, x, emb_table).reshape(batch, n, d)


# speedup vs baseline: 1.5156x; 1.0957x over previous
import jax
import jax.numpy as jnp
from jax.experimental import pallas as pl
from jax.experimental.pallas import tpu as pltpu

_SEQ_BLK = 1024


def _body(bounds_ref, x_ref, emb_ref, o_ref):
    j = pl.program_id(1)
    pos = jax.lax.broadcasted_iota(jnp.int32, (_SEQ_BLK, 1), 0) + j * _SEQ_BLK
    unknown_start = bounds_ref[0]
    back_start = bounds_ref[1]
    ignored_start = bounds_ref[2]
    t0 = emb_ref[0:1, :]
    t1 = emb_ref[1:2, :]
    t2 = emb_ref[2:3, :]
    emb = jnp.where(
        pos < unknown_start,
        t0,
        jnp.where(pos < back_start, t1, jnp.where(pos < ignored_start, t0, t2)),
    )
    o_ref[...] = x_ref[0] + emb


def kernel(x, emb_table, seq_len, front, back, keyframe_gap):
    batch, n, d = x.shape
    seq_len = jnp.asarray(seq_len, jnp.int32)
    front = jnp.asarray(front, jnp.int32)
    back = jnp.asarray(back, jnp.int32)
    keyframe_gap = jnp.asarray(keyframe_gap, jnp.int32)
    ignored_len = seq_len - front - back - keyframe_gap
    bounds = jnp.stack(
        [front, front + keyframe_gap, seq_len - ignored_len], axis=0
    ).astype(jnp.int32)

    grid = (batch, n // _SEQ_BLK)
    return pl.pallas_call(
        _body,
        grid=grid,
        in_specs=[
            pl.BlockSpec(memory_space=pltpu.SMEM),
            pl.BlockSpec((1, _SEQ_BLK, d), lambda b, j: (b, j, 0)),
            pl.BlockSpec((3, d), lambda b, j: (0, 0)),
        ],
        out_specs=pl.BlockSpec((_SEQ_BLK, d), lambda b, j: (b * (n // _SEQ_BLK) + j, 0)),
        out_shape=jax.ShapeDtypeStruct((batch * n, d), x.dtype),
    )(bounds, x, emb_table).reshape(batch, n, d)


# speedup vs baseline: 1.5868x; 1.0470x over previous
import jax
import jax.numpy as jnp
from jax.experimental import pallas as pl
from jax.experimental.pallas import tpu as pltpu

_SEQ_BLK = 2048


def _body(bounds_ref, x_ref, emb_ref, o_ref):
    j = pl.program_id(1)
    pos = jax.lax.broadcasted_iota(jnp.int32, (_SEQ_BLK, 1), 0) + j * _SEQ_BLK
    unknown_start = bounds_ref[0]
    back_start = bounds_ref[1]
    ignored_start = bounds_ref[2]
    t0 = emb_ref[0:1, :]
    t1 = emb_ref[1:2, :]
    t2 = emb_ref[2:3, :]
    emb = jnp.where(
        pos < unknown_start,
        t0,
        jnp.where(pos < back_start, t1, jnp.where(pos < ignored_start, t0, t2)),
    )
    o_ref[...] = x_ref[0] + emb


def kernel(x, emb_table, seq_len, front, back, keyframe_gap):
    batch, n, d = x.shape
    seq_len = jnp.asarray(seq_len, jnp.int32)
    front = jnp.asarray(front, jnp.int32)
    back = jnp.asarray(back, jnp.int32)
    keyframe_gap = jnp.asarray(keyframe_gap, jnp.int32)
    ignored_len = seq_len - front - back - keyframe_gap
    bounds = jnp.stack(
        [front, front + keyframe_gap, seq_len - ignored_len], axis=0
    ).astype(jnp.int32)

    grid = (batch, n // _SEQ_BLK)
    return pl.pallas_call(
        _body,
        grid=grid,
        in_specs=[
            pl.BlockSpec(memory_space=pltpu.SMEM),
            pl.BlockSpec((1, _SEQ_BLK, d), lambda b, j: (b, j, 0)),
            pl.BlockSpec((3, d), lambda b, j: (0, 0)),
        ],
        out_specs=pl.BlockSpec((_SEQ_BLK, d), lambda b, j: (b * (n // _SEQ_BLK) + j, 0)),
        out_shape=jax.ShapeDtypeStruct((batch * n, d), x.dtype),
    )(bounds, x, emb_table).reshape(batch, n, d)
